# Initial kernel scaffold; baseline (speedup 1.0000x reference)
import jax
import jax.numpy as jnp
from jax.experimental import pallas as pl


def kernel(Proto, feat, labels):
    raise NotImplementedError("write your pallas kernel here")


# speedup vs baseline: 4.7599x; 4.7599x over previous
import functools

import jax
import jax.numpy as jnp
from jax import lax
from jax.experimental import pallas as pl
from jax.experimental.pallas import tpu as pltpu
from jax.experimental.pallas import tpu_sc as plsc

C = 7
CPAD = 8
A = 128
LANES = 16
IGNORE = 255


def _sc_partials(feat, labels, n_workers, rows_per_w, chunk):
    n_chunks = rows_per_w // chunk
    mesh = plsc.VectorSubcoreMesh(core_axis_name="c", subcore_axis_name="s")

    @functools.partial(
        pl.kernel,
        mesh=mesh,
        out_type=(
            jax.ShapeDtypeStruct((n_workers, CPAD, A), jnp.float32),
            jax.ShapeDtypeStruct((n_workers, CPAD, LANES), jnp.float32),
        ),
        scratch_types=[
            pltpu.VMEM((chunk, A), jnp.float32),
            pltpu.VMEM((chunk, A), jnp.float32),
            pltpu.VMEM((chunk,), jnp.int32),
            pltpu.VMEM((chunk,), jnp.int32),
            pltpu.VMEM((CPAD, A), jnp.float32),
            pltpu.VMEM((CPAD, LANES), jnp.float32),
            pltpu.SemaphoreType.DMA,
            pltpu.SemaphoreType.DMA,
            pltpu.SemaphoreType.DMA,
            pltpu.SemaphoreType.DMA,
        ],
    )
    def body(feat_hbm, labels_hbm, sums_out, counts_out,
             fbuf0, fbuf1, lbuf0, lbuf1, acc, cacc, fsem0, fsem1, lsem0, lsem1):
        n_cores = 2
        wid = lax.axis_index("s") * n_cores + lax.axis_index("c")
        row0 = wid * rows_per_w

        fbufs = (fbuf0, fbuf1)
        lbufs = (lbuf0, lbuf1)
        fsems = (fsem0, fsem1)
        lsems = (lsem0, lsem1)

        def start(ci, slot):
            base = row0 + ci * chunk
            pltpu.make_async_copy(
                feat_hbm.at[pl.ds(base, chunk)], fbufs[slot], fsems[slot]
            ).start()
            pltpu.make_async_copy(
                labels_hbm.at[pl.ds(base, chunk)], lbufs[slot], lsems[slot]
            ).start()

        def wait(slot):
            pltpu.make_async_copy(
                feat_hbm.at[pl.ds(0, chunk)], fbufs[slot], fsems[slot]
            ).wait()
            pltpu.make_async_copy(
                labels_hbm.at[pl.ds(0, chunk)], lbufs[slot], lsems[slot]
            ).wait()

        zero = jnp.zeros((LANES,), jnp.float32)
        for r in range(CPAD):
            cacc[r, :] = zero
            for k in range(A // LANES):
                acc[r, pl.ds(k * LANES, LANES)] = zero

        ones = jnp.ones((LANES,), jnp.float32)

        def consume(slot):
            fbuf = fbufs[slot]
            lbuf = lbufs[slot]

            def group_body(g, _):
                lab16 = lbuf[pl.ds(g * LANES, LANES)]
                for r in range(LANES):
                    label = lab16[r]
                    valid = jnp.logical_and(label >= 0, label < C)
                    lab = jnp.where(valid, label, C)
                    row = g * LANES + r
                    for k in range(A // LANES):
                        plsc.addupdate(
                            acc.at[lab, pl.ds(k * LANES, LANES)],
                            fbuf[row, pl.ds(k * LANES, LANES)],
                        )
                    plsc.addupdate(cacc.at[lab], ones)
                return 0

            lax.fori_loop(0, chunk // LANES, group_body, 0)

        start(0, 0)
        n_pairs = n_chunks // 2

        def pair_body(p, _):
            ci0 = 2 * p
            wait(0)
            start(ci0 + 1, 1)
            consume(0)
            wait(1)

            @pl.when(ci0 + 2 < n_chunks)
            def _():
                start(ci0 + 2, 0)

            consume(1)
            return 0

        lax.fori_loop(0, n_pairs, pair_body, 0)
        if n_chunks % 2 == 1:
            wait(0)
            consume(0)

        pltpu.sync_copy(acc, sums_out.at[wid])
        pltpu.sync_copy(cacc, counts_out.at[wid])

    return body(feat, labels)


def _tc_finish_body(proto_ref, sums_ref, counts_ref, loss_ref, mean_ref):
    sums = jnp.sum(sums_ref[...], axis=0)
    counts = jnp.sum(counts_ref[...], axis=0)[:, :1]
    denom = jnp.where(counts == 0.0, 1.0, counts)
    mean = (sums / denom)[:C]
    mean_ref[...] = mean

    proto = proto_ref[...]
    pn = proto / jnp.maximum(
        jnp.sqrt(jnp.sum(proto * proto, axis=1, keepdims=True)), 1e-12)
    cn = mean / jnp.maximum(
        jnp.sqrt(jnp.sum(mean * mean, axis=1, keepdims=True)), 1e-12)
    logits = lax.dot_general(
        cn, pn, (((1,), (1,)), ((), ())), preferred_element_type=jnp.float32)

    row_sum = jnp.sum(logits, axis=1)
    valid = row_sum != 0.0
    m = jnp.max(logits, axis=1)
    lse = jnp.log(jnp.sum(jnp.exp(logits - m[:, None]), axis=1)) + m
    eye = (lax.broadcasted_iota(jnp.int32, (C, C), 0)
           == lax.broadcasted_iota(jnp.int32, (C, C), 1))
    diag = jnp.sum(jnp.where(eye, logits, 0.0), axis=1)
    ce = lse - diag
    num = jnp.sum(valid.astype(jnp.int32))
    loss = jnp.sum(jnp.where(valid, ce, 0.0)) / jnp.maximum(num, 1)
    loss_ref[...] = jnp.reshape(loss, (1, 1))


def kernel(Proto, feat, labels):
    n = feat.shape[0]
    n_workers = 32
    rows_per_w = n // n_workers
    chunk = 400

    sums, counts = _sc_partials(feat, labels, n_workers, rows_per_w, chunk)

    loss2d, mean = pl.pallas_call(
        _tc_finish_body,
        out_shape=(
            jax.ShapeDtypeStruct((1, 1), jnp.float32),
            jax.ShapeDtypeStruct((C, A), jnp.float32),
        ),
    )(Proto, sums, counts)
    return (loss2d[0, 0], mean)


# speedup vs baseline: 12.0241x; 2.5261x over previous
import functools

import jax
import jax.numpy as jnp
from jax import lax
from jax.experimental import pallas as pl
from jax.experimental.pallas import tpu as pltpu
from jax.experimental.pallas import tpu_sc as plsc

C = 7
CPAD = 8
A = 128
LANES = 16
IGNORE = 255


def _sc_partials(feat, labels, n_workers, rows_per_w, chunk):
    n_chunks = rows_per_w // chunk
    mesh = plsc.VectorSubcoreMesh(core_axis_name="c", subcore_axis_name="s")

    @functools.partial(
        pl.kernel,
        mesh=mesh,
        out_type=(
            jax.ShapeDtypeStruct((n_workers, CPAD, A), jnp.float32),
            jax.ShapeDtypeStruct((n_workers, CPAD, LANES), jnp.float32),
        ),
        scratch_types=[
            pltpu.VMEM((chunk, A), jnp.float32),
            pltpu.VMEM((chunk, A), jnp.float32),
            pltpu.VMEM((chunk,), jnp.int32),
            pltpu.VMEM((chunk,), jnp.int32),
            pltpu.VMEM((CPAD, A), jnp.float32),
            pltpu.VMEM((CPAD, LANES), jnp.float32),
            pltpu.SemaphoreType.DMA,
            pltpu.SemaphoreType.DMA,
            pltpu.SemaphoreType.DMA,
            pltpu.SemaphoreType.DMA,
        ],
    )
    def body(feat_hbm, labels_hbm, sums_out, counts_out,
             fbuf0, fbuf1, lbuf0, lbuf1, acc, cacc, fsem0, fsem1, lsem0, lsem1):
        n_cores = 2
        wid = lax.axis_index("s") * n_cores + lax.axis_index("c")
        row0 = wid * rows_per_w

        fbufs = (fbuf0, fbuf1)
        lbufs = (lbuf0, lbuf1)
        fsems = (fsem0, fsem1)
        lsems = (lsem0, lsem1)

        def start(ci, slot):
            base = row0 + ci * chunk
            pltpu.make_async_copy(
                feat_hbm.at[pl.ds(base, chunk)], fbufs[slot], fsems[slot]
            ).start()
            pltpu.make_async_copy(
                labels_hbm.at[pl.ds(base, chunk)], lbufs[slot], lsems[slot]
            ).start()

        def wait(slot):
            pltpu.make_async_copy(
                feat_hbm.at[pl.ds(0, chunk)], fbufs[slot], fsems[slot]
            ).wait()
            pltpu.make_async_copy(
                labels_hbm.at[pl.ds(0, chunk)], lbufs[slot], lsems[slot]
            ).wait()

        zero = jnp.zeros((LANES,), jnp.float32)
        for r in range(CPAD):
            cacc[r, :] = zero
            for k in range(A // LANES):
                acc[r, pl.ds(k * LANES, LANES)] = zero

        def consume(slot):
            fbuf = fbufs[slot]
            lbuf = lbufs[slot]

            def group_body(g, cnts):
                lab16 = lbuf[pl.ds(g * LANES, LANES)]
                one = jnp.ones((LANES,), jnp.float32)
                zero = jnp.zeros((LANES,), jnp.float32)
                cnts = tuple(
                    cnts[c] + jnp.where(lab16 == c, one, zero)
                    for c in range(C)
                )
                labs = []
                for r in range(LANES):
                    label = lab16[r]
                    valid = jnp.logical_and(label >= 0, label < C)
                    labs.append(jnp.where(valid, label, C))
                for r in range(LANES):
                    row = g * LANES + r
                    vals = [fbuf[row, pl.ds(k * LANES, LANES)]
                            for k in range(A // LANES)]
                    for k in range(A // LANES):
                        plsc.addupdate(
                            acc.at[labs[r], pl.ds(k * LANES, LANES)], vals[k])
                return cnts

            zcnt = jnp.zeros((LANES,), jnp.float32)
            cnts = lax.fori_loop(0, chunk // LANES, group_body, (zcnt,) * C)
            for c in range(C):
                plsc.addupdate(cacc.at[c], cnts[c])

        start(0, 0)
        n_pairs = n_chunks // 2

        def pair_body(p, _):
            ci0 = 2 * p
            wait(0)
            start(ci0 + 1, 1)
            consume(0)
            wait(1)

            @pl.when(ci0 + 2 < n_chunks)
            def _():
                start(ci0 + 2, 0)

            consume(1)
            return 0

        lax.fori_loop(0, n_pairs, pair_body, 0)
        if n_chunks % 2 == 1:
            wait(0)
            consume(0)

        pltpu.sync_copy(acc, sums_out.at[wid])
        pltpu.sync_copy(cacc, counts_out.at[wid])

    return body(feat, labels)


def _tc_finish_body(proto_ref, sums_ref, counts_ref, loss_ref, mean_ref):
    sums = jnp.sum(sums_ref[...], axis=0)
    counts = jnp.sum(counts_ref[...], axis=(0, 2))[:, None]
    denom = jnp.where(counts == 0.0, 1.0, counts)
    mean = (sums / denom)[:C]
    mean_ref[...] = mean

    proto = proto_ref[...]
    pn = proto / jnp.maximum(
        jnp.sqrt(jnp.sum(proto * proto, axis=1, keepdims=True)), 1e-12)
    cn = mean / jnp.maximum(
        jnp.sqrt(jnp.sum(mean * mean, axis=1, keepdims=True)), 1e-12)
    logits = lax.dot_general(
        cn, pn, (((1,), (1,)), ((), ())), preferred_element_type=jnp.float32)

    row_sum = jnp.sum(logits, axis=1)
    valid = row_sum != 0.0
    m = jnp.max(logits, axis=1)
    lse = jnp.log(jnp.sum(jnp.exp(logits - m[:, None]), axis=1)) + m
    eye = (lax.broadcasted_iota(jnp.int32, (C, C), 0)
           == lax.broadcasted_iota(jnp.int32, (C, C), 1))
    diag = jnp.sum(jnp.where(eye, logits, 0.0), axis=1)
    ce = lse - diag
    num = jnp.sum(valid.astype(jnp.int32))
    loss = jnp.sum(jnp.where(valid, ce, 0.0)) / jnp.maximum(num, 1)
    loss_ref[...] = jnp.reshape(loss, (1, 1))


def kernel(Proto, feat, labels):
    n = feat.shape[0]
    n_workers = 32
    rows_per_w = n // n_workers
    chunk = 400

    sums, counts = _sc_partials(feat, labels, n_workers, rows_per_w, chunk)

    loss2d, mean = pl.pallas_call(
        _tc_finish_body,
        out_shape=(
            jax.ShapeDtypeStruct((1, 1), jnp.float32),
            jax.ShapeDtypeStruct((C, A), jnp.float32),
        ),
    )(Proto, sums, counts)
    return (loss2d[0, 0], mean)


# speedup vs baseline: 13.1875x; 1.0968x over previous
import functools

import jax
import jax.numpy as jnp
from jax import lax
from jax.experimental import pallas as pl
from jax.experimental.pallas import tpu as pltpu
from jax.experimental.pallas import tpu_sc as plsc

C = 7
CPAD = 8
A = 128
LANES = 16
IGNORE = 255


def _sc_partials(feat, labels, n_workers, rows_per_w, chunk):
    n_chunks = rows_per_w // chunk
    mesh = plsc.VectorSubcoreMesh(core_axis_name="c", subcore_axis_name="s")

    batch = 80
    n_batch = chunk // batch
    n_sub = 16

    @functools.partial(
        pl.kernel,
        mesh=mesh,
        out_type=(
            jax.ShapeDtypeStruct((n_workers, CPAD, A), jnp.float32),
            jax.ShapeDtypeStruct((n_workers, CPAD, LANES), jnp.float32),
        ),
        scratch_types=[
            pltpu.VMEM((chunk, A), jnp.float32),
            pltpu.VMEM((chunk, A), jnp.float32),
            pltpu.VMEM((chunk,), jnp.int32),
            pltpu.VMEM((chunk,), jnp.int32),
            pltpu.VMEM((n_batch, batch), jnp.int32),
            pltpu.VMEM((n_batch, batch), jnp.int32),
            pltpu.VMEM((CPAD, A), jnp.float32),
            pltpu.VMEM((CPAD, LANES), jnp.float32),
            pltpu.VMEM_SHARED((n_sub * CPAD, A), jnp.float32),
            pltpu.SemaphoreType.DMA,
            pltpu.SemaphoreType.DMA,
            pltpu.SemaphoreType.DMA,
            pltpu.SemaphoreType.DMA,
            pltpu.SemaphoreType.DMA,
            pltpu.SemaphoreType.DMA,
        ],
    )
    def body(feat_hbm, labels_hbm, sums_out, counts_out,
             fbuf0, fbuf1, lbuf0, lbuf1, ibuf0, ibuf1, zbuf, cacc, shared,
             fsem0, fsem1, lsem0, lsem1, ssem0, ssem1):
        n_cores = 2
        sid = lax.axis_index("s")
        wid = sid * n_cores + lax.axis_index("c")
        row0 = wid * rows_per_w
        region0 = sid * CPAD

        fbufs = (fbuf0, fbuf1)
        lbufs = (lbuf0, lbuf1)
        ibufs = (ibuf0, ibuf1)
        fsems = (fsem0, fsem1)
        lsems = (lsem0, lsem1)
        ssems = (ssem0, ssem1)

        def start_in(ci, slot):
            base = row0 + ci * chunk
            pltpu.make_async_copy(
                feat_hbm.at[pl.ds(base, chunk)], fbufs[slot], fsems[slot]
            ).start()
            pltpu.make_async_copy(
                labels_hbm.at[pl.ds(base, chunk)], lbufs[slot], lsems[slot]
            ).start()

        def wait_in(slot):
            pltpu.make_async_copy(
                feat_hbm.at[pl.ds(0, chunk)], fbufs[slot], fsems[slot]
            ).wait()
            pltpu.make_async_copy(
                labels_hbm.at[pl.ds(0, chunk)], lbufs[slot], lsems[slot]
            ).wait()

        zero = jnp.zeros((LANES,), jnp.float32)
        for r in range(CPAD):
            for k in range(A // LANES):
                zbuf[r, pl.ds(k * LANES, LANES)] = zero
        pltpu.sync_copy(zbuf, shared.at[pl.ds(region0, CPAD)])

        base_vec = jnp.full((LANES,), CPAD, dtype=jnp.int32) * sid
        sevenv = jnp.full((LANES,), C, dtype=jnp.uint32)
        one = jnp.ones((LANES,), jnp.float32)
        zf = jnp.zeros((LANES,), jnp.float32)

        def scat(slot, j):
            return pltpu.async_copy(
                fbufs[slot].at[pl.ds(j * batch, batch)],
                shared.at[ibufs[slot].at[j]],
                ssems[slot],
                add=True,
            )

        def consume(slot, cnts):
            lbuf = lbufs[slot]
            ibuf = ibufs[slot]
            for g in range(chunk // LANES):
                lab16 = lbuf[pl.ds(g * LANES, LANES)]
                cnts = tuple(
                    cnts[c] + jnp.where(lab16 == c, one, zf)
                    for c in range(C)
                )
                lab_u = jnp.minimum(
                    lab16.astype(jnp.uint32), sevenv).astype(jnp.int32)
                idx = lab_u + base_vec
                b = (g * LANES) // batch
                off = (g * LANES) % batch
                ibuf[b, pl.ds(off, LANES)] = idx
            descs = [scat(slot, j) for j in range(n_batch)]
            for d in descs:
                d.wait()
            return cnts

        start_in(0, 0)
        start_in(1, 1)
        n_pairs = n_chunks // 2
        zcnt = jnp.zeros((LANES,), jnp.float32)

        def pair_body(p, cnts):
            ci0 = 2 * p
            wait_in(0)
            cnts = consume(0, cnts)

            @pl.when(ci0 + 2 < n_chunks)
            def _():
                start_in(ci0 + 2, 0)

            wait_in(1)
            cnts = consume(1, cnts)

            @pl.when(ci0 + 3 < n_chunks)
            def _():
                start_in(ci0 + 3, 1)

            return cnts

        cnts = lax.fori_loop(0, n_pairs, pair_body, (zcnt,) * C)
        if n_chunks % 2 == 1:
            wait_in(0)
            cnts = consume(0, cnts)

        for c in range(C):
            cacc[c, :] = cnts[c]
        cacc[C, :] = zf

        pltpu.sync_copy(shared.at[pl.ds(region0, CPAD)], sums_out.at[wid])
        pltpu.sync_copy(cacc, counts_out.at[wid])

    return body(feat, labels)


def _tc_finish_body(proto_ref, sums_ref, counts_ref, loss_ref, mean_ref):
    sums = jnp.sum(sums_ref[...], axis=0)
    counts = jnp.sum(counts_ref[...], axis=(0, 2))[:, None]
    denom = jnp.where(counts == 0.0, 1.0, counts)
    mean = (sums / denom)[:C]
    mean_ref[...] = mean

    proto = proto_ref[...]
    pn = proto / jnp.maximum(
        jnp.sqrt(jnp.sum(proto * proto, axis=1, keepdims=True)), 1e-12)
    cn = mean / jnp.maximum(
        jnp.sqrt(jnp.sum(mean * mean, axis=1, keepdims=True)), 1e-12)
    logits = lax.dot_general(
        cn, pn, (((1,), (1,)), ((), ())), preferred_element_type=jnp.float32)

    row_sum = jnp.sum(logits, axis=1)
    valid = row_sum != 0.0
    m = jnp.max(logits, axis=1)
    lse = jnp.log(jnp.sum(jnp.exp(logits - m[:, None]), axis=1)) + m
    eye = (lax.broadcasted_iota(jnp.int32, (C, C), 0)
           == lax.broadcasted_iota(jnp.int32, (C, C), 1))
    diag = jnp.sum(jnp.where(eye, logits, 0.0), axis=1)
    ce = lse - diag
    num = jnp.sum(valid.astype(jnp.int32))
    loss = jnp.sum(jnp.where(valid, ce, 0.0)) / jnp.maximum(num, 1)
    loss_ref[...] = jnp.reshape(loss, (1, 1))


def kernel(Proto, feat, labels):
    n = feat.shape[0]
    n_workers = 32
    rows_per_w = n // n_workers
    chunk = 400

    sums, counts = _sc_partials(feat, labels, n_workers, rows_per_w, chunk)

    loss2d, mean = pl.pallas_call(
        _tc_finish_body,
        out_shape=(
            jax.ShapeDtypeStruct((1, 1), jnp.float32),
            jax.ShapeDtypeStruct((C, A), jnp.float32),
        ),
    )(Proto, sums, counts)
    return (loss2d[0, 0], mean)
